# Initial kernel scaffold; baseline (speedup 1.0000x reference)
import jax
import jax.numpy as jnp
from jax.experimental import pallas as pl


def kernel(x, g, b, Wqkv, Wout, bout):
    raise NotImplementedError("write your pallas kernel here")


# speedup vs baseline: 4.2398x; 4.2398x over previous
import functools

import jax
import jax.numpy as jnp
import numpy as np
from jax import lax
from jax.experimental import pallas as pl
from jax.experimental.pallas import tpu as pltpu
from jax.experimental.pallas import tpu_sc as plsc

DIM = 256
HEADS = 8
DIM_HEAD = 32
INNER = HEADS * DIM_HEAD
DTK, HTK, WTK = 4, 8, 8
EPS = 1e-5
D0, H0, W0 = 16, 32, 32
N = D0 * H0 * W0
NSEL = DTK * HTK * WTK


def _k1_body(x_ref, w_ref, g_ref, b_ref, q_ref, k_ref, v_ref,
             qpd_ref, qph_ref, qpw_ref, ksd_ref, kh_ref):
    d = pl.program_id(1)
    xb = x_ref[0, :, :]
    mean = jnp.mean(xb, axis=0, keepdims=True)
    xc = xb - mean
    var = jnp.mean(xc * xc, axis=0, keepdims=True)
    xn = xc / jnp.sqrt(var + EPS) * g_ref[...] + b_ref[...]
    qkvt = jax.lax.dot_general(
        xn.astype(jnp.bfloat16), w_ref[...].astype(jnp.bfloat16),
        (((0,), (0,)), ((), ())),
        preferred_element_type=jnp.float32)
    qt = qkvt[:, :DIM]
    kt = qkvt[:, DIM:2 * DIM]
    vt = qkvt[:, 2 * DIM:]
    r = jax.lax.broadcasted_iota(jnp.int32, (DIM, HEADS), 0)
    c = jax.lax.broadcasted_iota(jnp.int32, (DIM, HEADS), 1)
    G = (r // DIM_HEAD == c).astype(jnp.float32)

    def l2n(t):
        ssq = jax.lax.dot_general(t * t, G, (((1,), (0,)), ((), ())),
                                  precision=jax.lax.Precision.HIGHEST,
                                  preferred_element_type=jnp.float32)
        nrm = jax.lax.dot_general(jnp.sqrt(ssq), G, (((1,), (1,)), ((), ())),
                                  precision=jax.lax.Precision.HIGHEST,
                                  preferred_element_type=jnp.float32)
        return t / jnp.maximum(nrm, 1e-12)

    qn = l2n(qt)
    kn = l2n(kt)
    q_ref[0, :, :] = qn.astype(jnp.bfloat16)
    k_ref[0, :, :] = kn
    v_ref[0, :, :] = vt
    aq = jnp.abs(qn)
    ak = jnp.abs(kn)
    qpd_ref[0, 0, 0, :] = jnp.sum(aq, axis=0)
    ksd_ref[0, 0, 0, :] = jnp.sum(ak, axis=0)
    aq3 = aq.reshape(H0, W0, DIM)
    ak3 = ak.reshape(H0, W0, DIM)
    ph = jnp.sum(aq3, axis=1)
    pw = jnp.sum(aq3, axis=0)
    kh_ref[0, 0, :, :] = jnp.sum(ak3, axis=1)

    @pl.when(d == 0)
    def _():
        qph_ref[0, :, :] = ph
        qpw_ref[0, :, :] = pw

    @pl.when(d != 0)
    def _():
        qph_ref[0, :, :] += ph
        qpw_ref[0, :, :] += pw


def _run_k1(x3, Wq2, gcol, bcol):
    return pl.pallas_call(
        _k1_body,
        grid=(2, D0),
        in_specs=[
            pl.BlockSpec((1, DIM, H0 * W0), lambda b, d: (b, 0, d)),
            pl.BlockSpec((DIM, 3 * INNER), lambda b, d: (0, 0)),
            pl.BlockSpec((DIM, 1), lambda b, d: (0, 0)),
            pl.BlockSpec((DIM, 1), lambda b, d: (0, 0)),
        ],
        out_specs=[
            pl.BlockSpec((1, H0 * W0, DIM), lambda b, d: (b, d, 0)),
            pl.BlockSpec((1, H0 * W0, DIM), lambda b, d: (b, d, 0)),
            pl.BlockSpec((1, H0 * W0, DIM), lambda b, d: (b, d, 0)),
            pl.BlockSpec((1, 1, 1, DIM), lambda b, d: (b, d, 0, 0)),
            pl.BlockSpec((1, H0, DIM), lambda b, d: (b, 0, 0)),
            pl.BlockSpec((1, W0, DIM), lambda b, d: (b, 0, 0)),
            pl.BlockSpec((1, 1, 1, DIM), lambda b, d: (b, d, 0, 0)),
            pl.BlockSpec((1, 1, H0, DIM), lambda b, d: (b, d, 0, 0)),
        ],
        out_shape=[
            jax.ShapeDtypeStruct((2, N, DIM), jnp.bfloat16),
            jax.ShapeDtypeStruct((2, N, DIM), jnp.float32),
            jax.ShapeDtypeStruct((2, N, DIM), jnp.float32),
            jax.ShapeDtypeStruct((2, D0, 1, DIM), jnp.float32),
            jax.ShapeDtypeStruct((2, H0, DIM), jnp.float32),
            jax.ShapeDtypeStruct((2, W0, DIM), jnp.float32),
            jax.ShapeDtypeStruct((2, D0, 1, DIM), jnp.float32),
            jax.ShapeDtypeStruct((2, D0, H0, DIM), jnp.float32),
        ],
    )(x3, Wq2, gcol, bcol)


def _sc_select_gather(qpd8, ksd8, qph8, qpw8, kh8, kn8, v8):
    info = plsc.get_sparse_core_info()
    NC = info.num_cores
    mesh = plsc.VectorSubcoreMesh(core_axis_name="c", subcore_axis_name="s")

    @functools.partial(
        pl.kernel, mesh=mesh,
        compiler_params=pltpu.CompilerParams(needs_layout_passes=False),
        out_type=[jax.ShapeDtypeStruct((2 * HEADS, NSEL, DIM_HEAD), jnp.float32),
                  jax.ShapeDtypeStruct((2 * HEADS, NSEL, DIM_HEAD), jnp.float32)],
        scratch_types=[
            pltpu.VMEM((D0, DIM_HEAD), jnp.float32),
            pltpu.VMEM((D0, DIM_HEAD), jnp.float32),
            pltpu.VMEM((H0, DIM_HEAD), jnp.float32),
            pltpu.VMEM((W0, DIM_HEAD), jnp.float32),
            pltpu.VMEM((H0, DIM_HEAD), jnp.float32),
            pltpu.VMEM((128, DIM), jnp.float32),
            pltpu.VMEM((W0, DIM_HEAD), jnp.float32),
            pltpu.VMEM((128,), jnp.int32),
            pltpu.VMEM((1024,), jnp.int32),
            pltpu.VMEM((NSEL,), jnp.int32),
            pltpu.VMEM((NSEL, DIM_HEAD), jnp.float32),
            pltpu.VMEM((NSEL, DIM_HEAD), jnp.float32),
            pltpu.SemaphoreType.DMA,
        ],
    )
    def sc_kernel(qpd_h, ksd_h, qph_h, qpw_h, kh_h, kn_h, v_h,
                  ksel_o, vsel_o,
                  qpd_v, ksd_v, qph_v, qpw_v, ksh_v, knw_v, ksw_v,
                  idx128_v, idx1k_v, kidx_v, ksel_v, vsel_v, sem):
        wid = lax.axis_index("s") * NC + lax.axis_index("c")

        @pl.when(wid < 2 * HEADS)
        def _():
            b = wid // HEADS
            m = wid % HEADS
            i16 = lax.iota(jnp.int32, 16)
            zero16 = jnp.zeros((16,), jnp.float32)

            def row_scores(qv_ref, kv_ref, nrows, absk=False):
                chunks = []
                for base in range(0, nrows, 16):
                    sv = zero16
                    for rr in range(16):
                        row = base + rr
                        kv0 = kv_ref[row, pl.ds(0, 16)]
                        kv1 = kv_ref[row, pl.ds(16, 16)]
                        if absk:
                            kv0 = jnp.abs(kv0)
                            kv1 = jnp.abs(kv1)
                        acc = (qv_ref[row, pl.ds(0, 16)] * kv0
                               + qv_ref[row, pl.ds(16, 16)] * kv1)
                        sv = jnp.where(i16 == rr, jnp.sum(acc), sv)
                    chunks.append(sv)
                return chunks

            pltpu.sync_copy(qpd_h.at[b, :, m, :], qpd_v)
            pltpu.sync_copy(ksd_h.at[b, :, m, :], ksd_v)
            (sd,) = row_scores(qpd_v, ksd_v, D0)
            _, dsel = plsc.sort_key_val(sd, i16, descending=True)

            for jd in range(DTK):
                d_s = dsel[jd]
                for ch in range(2):
                    rows = (b * D0 + d_s) * H0 + ch * 16 + i16
                    idx128_v[pl.ds((jd * 2 + ch) * 16, 16)] = rows
            pltpu.async_copy(kh_h.at[idx128_v], knw_v, sem).wait()
            for h in range(H0):
                ksh_v[h, pl.ds(0, 16)] = zero16
                ksh_v[h, pl.ds(16, 16)] = zero16

            def accum_h(r, carry):
                h = r % H0
                for cc in range(2):
                    plsc.addupdate(
                        ksh_v.at[h, pl.ds(cc * 16, 16)],
                        knw_v[r, pl.ds(m * DIM_HEAD + cc * 16, 16)])
                return carry
            lax.fori_loop(0, 128, accum_h, 0)

            pltpu.sync_copy(qph_h.at[b, :, m, :], qph_v)
            sh0, sh1 = row_scores(qph_v, ksh_v, H0)

            def top8(s0, s1):
                k0, v0 = plsc.sort_key_val(s0, i16, descending=True)
                k1, v1 = plsc.sort_key_val(s1, i16 + 16, descending=True)
                ck, cv = k0, v0
                for l in range(8, 16):
                    ck = jnp.where(i16 == l, k1[l - 8], ck)
                    cv = jnp.where(i16 == l, v1[l - 8], cv)
                _, cvs = plsc.sort_key_val(ck, cv, descending=True)
                return cvs

            hsel = top8(sh0, sh1)

            for jd in range(DTK):
                d_s = dsel[jd]
                for jh in range(HTK):
                    h_s = hsel[jh]
                    base = b * N + d_s * (H0 * W0) + h_s * W0
                    for ch in range(2):
                        rows = base + ch * 16 + i16
                        idx1k_v[pl.ds(((jd * HTK + jh) * 2 + ch) * 16, 16)] = rows
            for w in range(W0):
                ksw_v[w, pl.ds(0, 16)] = zero16
                ksw_v[w, pl.ds(16, 16)] = zero16

            def accum_w(r, carry):
                w = r % W0
                for cc in range(2):
                    plsc.addupdate(
                        ksw_v.at[w, pl.ds(cc * 16, 16)],
                        jnp.abs(knw_v[r, pl.ds(m * DIM_HEAD + cc * 16, 16)]))
                return carry
            for j in range(8):
                pltpu.async_copy(kn_h.at[idx1k_v.at[pl.ds(j * 128, 128)]],
                                 knw_v, sem).wait()
                lax.fori_loop(0, 128, accum_w, 0)

            pltpu.sync_copy(qpw_h.at[b, :, m, :], qpw_v)
            sw0, sw1 = row_scores(qpw_v, ksw_v, W0)
            wsel = top8(sw0, sw1)

            wv2 = jnp.zeros((16,), jnp.int32)
            for l in range(16):
                wv2 = jnp.where(i16 == l, wsel[l % WTK], wv2)
            for jd in range(DTK):
                d_s = dsel[jd]
                for jh2 in range(HTK // 2):
                    h0 = hsel[jh2 * 2]
                    h1 = hsel[jh2 * 2 + 1]
                    hvec = jnp.where(i16 < 8, jnp.full((16,), h0, jnp.int32),
                                     jnp.full((16,), h1, jnp.int32))
                    rows = b * N + d_s * (H0 * W0) + hvec * W0 + wv2
                    kidx_v[pl.ds((jd * 4 + jh2) * 16, 16)] = rows
            for j in range(2):
                def extract(r, carry, _dst=None, _base=j * 128):
                    for cc in range(2):
                        carry_ref = _dst
                        carry_ref[_base + r, pl.ds(cc * 16, 16)] = (
                            knw_v[r, pl.ds(m * DIM_HEAD + cc * 16, 16)])
                    return carry
                pltpu.async_copy(kn_h.at[kidx_v.at[pl.ds(j * 128, 128)]],
                                 knw_v, sem).wait()
                lax.fori_loop(0, 128, functools.partial(extract, _dst=ksel_v), 0)
                pltpu.async_copy(v_h.at[kidx_v.at[pl.ds(j * 128, 128)]],
                                 knw_v, sem).wait()
                lax.fori_loop(0, 128, functools.partial(extract, _dst=vsel_v), 0)
            pltpu.sync_copy(ksel_v, ksel_o.at[wid])
            pltpu.sync_copy(vsel_v, vsel_o.at[wid])

    return sc_kernel(qpd8, ksd8, qph8, qpw8, kh8, kn8, v8)


def _select_and_gather(qpd, qph, qpw, ksd, KH, kn, v):
    BH = 2 * HEADS
    qpd_h = qpd.reshape(2, D0, HEADS, DIM_HEAD).transpose(0, 2, 1, 3)
    ksd_h = ksd.reshape(2, D0, HEADS, DIM_HEAD).transpose(0, 2, 1, 3)
    score_d = jnp.einsum('bhdc,bhdc->bhd', qpd_h, ksd_h).reshape(BH, D0)
    _, idx_d = jax.lax.top_k(score_d, DTK)

    KH_h = KH.reshape(2, D0, H0, HEADS, DIM_HEAD)
    mask_d = jnp.zeros((BH, D0), jnp.float32).at[
        jnp.arange(BH)[:, None], idx_d].set(1.0)
    mdr = mask_d.reshape(2, HEADS, D0)
    ks_h = jnp.einsum('bmd,bdhmc->bmhc', mdr, KH_h)
    qph_h = qph.reshape(2, H0, HEADS, DIM_HEAD).transpose(0, 2, 1, 3)
    score_h = jnp.einsum('bmhc,bmhc->bmh', qph_h, ks_h).reshape(BH, H0)
    _, idx_h = jax.lax.top_k(score_h, HTK)
    mask_h = jnp.zeros((BH, H0), jnp.float32).at[
        jnp.arange(BH)[:, None], idx_h].set(1.0)
    mhr = mask_h.reshape(2, HEADS, H0)

    ka = jnp.abs(kn).reshape(2, D0, H0, W0, HEADS, DIM_HEAD)
    ks_w = jnp.einsum('bmd,bmh,bdhwmc->bmwc', mdr, mhr, ka)
    qpw_h = qpw.reshape(2, W0, HEADS, DIM_HEAD).transpose(0, 2, 1, 3)
    score_w = jnp.einsum('bmwc,bmwc->bmw', qpw_h, ks_w).reshape(BH, W0)
    _, idx_w = jax.lax.top_k(score_w, WTK)

    kpos = (idx_d[:, :, None, None] * (H0 * W0)
            + idx_h[:, None, :, None] * W0
            + idx_w[:, None, None, :]).reshape(BH, NSEL)
    kn_h = kn.reshape(2, N, HEADS, DIM_HEAD)
    v_h = v.reshape(2, N, HEADS, DIM_HEAD)
    b_ix = (jnp.arange(BH) // HEADS)[:, None]
    h_ix = (jnp.arange(BH) % HEADS)[:, None]
    k_sel = kn_h[b_ix, kpos, h_ix]
    v_sel = v_h[b_ix, kpos, h_ix]
    return k_sel, v_sel


def _k3_body(q_ref, k_ref, v_ref, o_ref):
    q_all = q_ref[0, :, :]
    for h in range(HEADS):
        q = q_all[:, h * DIM_HEAD:(h + 1) * DIM_HEAD]
        k = k_ref[0, h, :, :]
        v = v_ref[0, h, :, :]
        s = jax.lax.dot_general(q, k.astype(jnp.bfloat16),
                                (((1,), (1,)), ((), ())),
                                preferred_element_type=jnp.float32)
        m = jnp.max(s, axis=1, keepdims=True)
        e = jnp.exp(s - m)
        p = e / jnp.sum(e, axis=1, keepdims=True)
        o_ref[0, h, :, :] = jax.lax.dot_general(
            p.astype(jnp.bfloat16), v.astype(jnp.bfloat16),
            (((1,), (0,)), ((), ())),
            preferred_element_type=jnp.float32).astype(jnp.bfloat16)


def _run_k3(qn, k_sel, v_sel, tq=2048):
    return pl.pallas_call(
        _k3_body,
        grid=(2, N // tq),
        in_specs=[
            pl.BlockSpec((1, tq, DIM), lambda b, t: (b, t, 0)),
            pl.BlockSpec((1, HEADS, NSEL, DIM_HEAD), lambda b, t: (b, 0, 0, 0)),
            pl.BlockSpec((1, HEADS, NSEL, DIM_HEAD), lambda b, t: (b, 0, 0, 0)),
        ],
        out_specs=pl.BlockSpec((1, HEADS, tq, DIM_HEAD),
                               lambda b, t: (b, 0, t, 0)),
        out_shape=jax.ShapeDtypeStruct((2, HEADS, N, DIM_HEAD), jnp.bfloat16),
    )(qn, k_sel.reshape(2, HEADS, NSEL, DIM_HEAD),
      v_sel.reshape(2, HEADS, NSEL, DIM_HEAD))


def _k4_body(y_ref, w_ref, bout_ref, o_ref):
    y = y_ref[0].reshape(DIM, H0 * W0)
    o_ref[0, :, :] = jax.lax.dot_general(
        w_ref[...].astype(jnp.bfloat16), y,
        (((1,), (0,)), ((), ())),
        preferred_element_type=jnp.float32) + bout_ref[...]


def _run_k4(of5, Wout, bout2):
    return pl.pallas_call(
        _k4_body,
        grid=(2, D0),
        in_specs=[
            pl.BlockSpec((1, HEADS, DIM_HEAD, H0, W0),
                         lambda b, d: (b, 0, 0, d, 0)),
            pl.BlockSpec((DIM, DIM), lambda b, d: (0, 0)),
            pl.BlockSpec((DIM, 1), lambda b, d: (0, 0)),
        ],
        out_specs=pl.BlockSpec((1, DIM, H0 * W0), lambda b, d: (b, 0, d)),
        out_shape=jax.ShapeDtypeStruct((2, DIM, N), jnp.float32),
    )(of5, Wout, bout2)


def kernel(x, g, b, Wqkv, Wout, bout):
    Wq2 = Wqkv.T
    x3 = x.reshape(2, DIM, N)
    qn, kn, v, qpd, qph, qpw, ksd, KH = _run_k1(
        x3, Wq2, g.reshape(DIM, 1), b.reshape(DIM, 1))
    k_sel, v_sel = _sc_select_gather(
        qpd.reshape(2, D0, HEADS, DIM_HEAD),
        ksd.reshape(2, D0, HEADS, DIM_HEAD),
        qph.reshape(2, H0, HEADS, DIM_HEAD),
        qpw.reshape(2, W0, HEADS, DIM_HEAD),
        KH.reshape(2 * D0 * H0, DIM),
        kn.reshape(2 * N, DIM),
        v.reshape(2 * N, DIM))
    out_flat = _run_k3(qn, k_sel, v_sel)
    of5 = out_flat.reshape(2, HEADS, DIM_HEAD, H0 * D0, W0)
    out = _run_k4(of5, Wout, bout.reshape(DIM, 1))
    return out.reshape(2, DIM, D0, H0, W0)
